# baseline (device time: 61317 ns/iter reference)
import functools

import jax
import jax.numpy as jnp
from jax import lax
from jax.experimental import pallas as pl
from jax.experimental.pallas import tpu as pltpu

N_CHUNKS = 16


def kernel(x):
    m, n = x.shape
    half = m // 2
    rows = half // N_CHUNKS

    def body(x_hbm, out_hbm, xh_ref, send_ref, recv_y_ref, sum_ref,
             copy_sems, store_sems, s1_send, s1_recv, s2_send, s2_recv):
        my_x = lax.axis_index("x")
        my_y = lax.axis_index("y")
        h0 = my_x * half
        y_nbr = (my_x, 1 - my_y)
        x_nbr = (1 - my_x, my_y)
        peers = [x_nbr, y_nbr, (1 - my_x, 1 - my_y)]

        def ds(i):
            return (pl.ds(i * rows, rows), slice(None))

        def ds_out(i):
            return (pl.ds(h0 + i * rows, rows), slice(None))

        barrier = pltpu.get_barrier_semaphore()
        for dev in peers:
            pl.semaphore_signal(barrier, inc=1, device_id=dev,
                                device_id_type=pl.DeviceIdType.MESH)
        pl.semaphore_wait(barrier, 3)

        copies = []
        for i in range(N_CHUNKS):
            cp = pltpu.make_async_copy(
                x_hbm.at[pl.ds(h0 + i * rows, rows), :],
                xh_ref.at[ds(i)], copy_sems.at[i])
            cp.start()
            copies.append(cp)

        rdma1 = []
        for i in range(N_CHUNKS):
            copies[i].wait()
            send_ref[ds(i)] = xh_ref[ds(i)].astype(jnp.bfloat16)
            r = pltpu.make_async_remote_copy(
                src_ref=send_ref.at[ds(i)], dst_ref=recv_y_ref.at[ds(i)],
                send_sem=s1_send.at[i], recv_sem=s1_recv.at[i],
                device_id=y_nbr, device_id_type=pl.DeviceIdType.MESH)
            r.start()
            rdma1.append(r)

        rdma2 = []
        stores = []
        for i in range(N_CHUNKS):
            rdma1[i].wait_recv()
            s32 = xh_ref[ds(i)] + recv_y_ref[ds(i)].astype(jnp.float32)
            sum_ref[ds(i)] = s32.astype(jnp.bfloat16)
            st = pltpu.make_async_copy(
                sum_ref.at[ds(i)], out_hbm.at[ds_out(i)], store_sems.at[i])
            st.start()
            stores.append(st)
            r = pltpu.make_async_remote_copy(
                src_ref=sum_ref.at[ds(i)], dst_ref=out_hbm.at[ds_out(i)],
                send_sem=s2_send.at[i], recv_sem=s2_recv.at[i],
                device_id=x_nbr, device_id_type=pl.DeviceIdType.MESH)
            r.start()
            rdma2.append(r)

        for i in range(N_CHUNKS):
            rdma2[i].wait_recv()
        for i in range(N_CHUNKS):
            stores[i].wait()
            rdma1[i].wait_send()
            rdma2[i].wait_send()

        @functools.partial(pl.run_scoped, sem2=pltpu.SemaphoreType.REGULAR)
        def _(sem2):
            for dev in peers:
                pl.semaphore_signal(sem2, inc=1, device_id=dev,
                                    device_id_type=pl.DeviceIdType.MESH)
            pl.semaphore_wait(sem2, 3)

    return pl.pallas_call(
        body,
        out_shape=jax.ShapeDtypeStruct((m, n), jnp.bfloat16),
        in_specs=[pl.BlockSpec(memory_space=pl.ANY)],
        out_specs=pl.BlockSpec(memory_space=pl.ANY),
        scratch_shapes=[
            pltpu.VMEM((half, n), jnp.float32),
            pltpu.VMEM((half, n), jnp.bfloat16),
            pltpu.VMEM((half, n), jnp.bfloat16),
            pltpu.VMEM((half, n), jnp.bfloat16),
            pltpu.SemaphoreType.DMA((N_CHUNKS,)),
            pltpu.SemaphoreType.DMA((N_CHUNKS,)),
            pltpu.SemaphoreType.DMA((N_CHUNKS,)),
            pltpu.SemaphoreType.DMA((N_CHUNKS,)),
            pltpu.SemaphoreType.DMA((N_CHUNKS,)),
            pltpu.SemaphoreType.DMA((N_CHUNKS,)),
        ],
        compiler_params=pltpu.CompilerParams(collective_id=0),
    )(x)


# device time: 56465 ns/iter; 1.0859x vs baseline; 1.0859x over previous
import functools
import os

import jax
import jax.numpy as jnp
from jax import lax
from jax.experimental import pallas as pl
from jax.experimental.pallas import tpu as pltpu

N_CHUNKS = int(os.environ.get("SCBAND_NCHUNKS", "16"))
_VARIANT = os.environ.get("SCBAND_VARIANT", "full")


def kernel(x):
    m, n = x.shape
    half = m // 2
    rows = half // N_CHUNKS

    def body(x_hbm, out_hbm, xh_ref, send_ref, recv_y_ref, sum_ref,
             copy_sems, store_sems, s1_send, s1_recv, s2_send, s2_recv):
        my_x = lax.axis_index("x")
        my_y = lax.axis_index("y")
        h0 = my_x * half
        y_nbr = (my_x, 1 - my_y)
        x_nbr = (1 - my_x, my_y)
        peers = [x_nbr, y_nbr, (1 - my_x, 1 - my_y)]

        def ds(i):
            return (pl.ds(i * rows, rows), slice(None))

        def ds_out(i):
            return (pl.ds(h0 + i * rows, rows), slice(None))

        barrier = pltpu.get_barrier_semaphore()
        for dev in peers:
            pl.semaphore_signal(barrier, inc=1, device_id=dev,
                                device_id_type=pl.DeviceIdType.MESH)
        pl.semaphore_wait(barrier, 3)

        copies = []
        for i in range(N_CHUNKS):
            cp = pltpu.make_async_copy(
                x_hbm.at[pl.ds(h0 + i * rows, rows), :],
                xh_ref.at[ds(i)], copy_sems.at[i])
            cp.start()
            copies.append(cp)

        rdma1 = []
        for i in range(N_CHUNKS):
            copies[i].wait()
            send_ref[ds(i)] = xh_ref[ds(i)].astype(jnp.bfloat16)
            if _VARIANT == "phase2":
                continue
            r = pltpu.make_async_remote_copy(
                src_ref=send_ref.at[ds(i)], dst_ref=recv_y_ref.at[ds(i)],
                send_sem=s1_send.at[i], recv_sem=s1_recv.at[i],
                device_id=y_nbr, device_id_type=pl.DeviceIdType.MESH)
            r.start()
            rdma1.append(r)

        rdma2 = []
        stores = []
        for i in range(N_CHUNKS):
            if _VARIANT != "phase2":
                rdma1[i].wait_recv()
            s32 = xh_ref[ds(i)] + recv_y_ref[ds(i)].astype(jnp.float32)
            sum_ref[ds(i)] = s32.astype(jnp.bfloat16)
            st = pltpu.make_async_copy(
                sum_ref.at[ds(i)], out_hbm.at[ds_out(i)], store_sems.at[i])
            st.start()
            stores.append(st)
            if _VARIANT == "phase1":
                continue
            r = pltpu.make_async_remote_copy(
                src_ref=sum_ref.at[ds(i)], dst_ref=out_hbm.at[ds_out(i)],
                send_sem=s2_send.at[i], recv_sem=s2_recv.at[i],
                device_id=x_nbr, device_id_type=pl.DeviceIdType.MESH)
            r.start()
            rdma2.append(r)

        for r in rdma2:
            r.wait_recv()
        for i in range(N_CHUNKS):
            stores[i].wait()
        for r in rdma1:
            r.wait_send()
        for r in rdma2:
            r.wait_send()

        @functools.partial(pl.run_scoped, sem2=pltpu.SemaphoreType.REGULAR)
        def _(sem2):
            for dev in peers:
                pl.semaphore_signal(sem2, inc=1, device_id=dev,
                                    device_id_type=pl.DeviceIdType.MESH)
            pl.semaphore_wait(sem2, 3)

    return pl.pallas_call(
        body,
        out_shape=jax.ShapeDtypeStruct((m, n), jnp.bfloat16),
        in_specs=[pl.BlockSpec(memory_space=pl.ANY)],
        out_specs=pl.BlockSpec(memory_space=pl.ANY),
        scratch_shapes=[
            pltpu.VMEM((half, n), jnp.float32),
            pltpu.VMEM((half, n), jnp.bfloat16),
            pltpu.VMEM((half, n), jnp.bfloat16),
            pltpu.VMEM((half, n), jnp.bfloat16),
            pltpu.SemaphoreType.DMA((N_CHUNKS,)),
            pltpu.SemaphoreType.DMA((N_CHUNKS,)),
            pltpu.SemaphoreType.DMA((N_CHUNKS,)),
            pltpu.SemaphoreType.DMA((N_CHUNKS,)),
            pltpu.SemaphoreType.DMA((N_CHUNKS,)),
            pltpu.SemaphoreType.DMA((N_CHUNKS,)),
        ],
        compiler_params=pltpu.CompilerParams(collective_id=0),
    )(x)
